# baseline (speedup 1.0000x reference)
import jax
import jax.numpy as jnp
from jax import lax
from jax.experimental import pallas as pl
from jax.experimental.pallas import tpu as pltpu

_EPS = 1e-12
_BF = jnp.bfloat16
_CP = pltpu.CompilerParams(dimension_semantics=("arbitrary",),
                           vmem_limit_bytes=60 * 1024 * 1024)


def _pass_a(x_ref, h_ref, wcol_ref, wrow_ref, w1_ref, b1_ref,
            isdv_ref, srow_ref, e1_ref, u_scr):
    i = pl.program_id(0)
    nsteps = pl.num_programs(0)
    nb = h_ref.shape[0]
    hb = h_ref[...]
    dv = jnp.dot(hb, wcol_ref[...],
                 preferred_element_type=jnp.float32)
    isdv = lax.rsqrt(dv + _EPS)
    isdv_ref[...] = isdv
    xw = jnp.dot(x_ref[...], w1_ref[...],
                 preferred_element_type=jnp.float32) + b1_ref[...]
    x1 = (xw * isdv).astype(_BF)
    lhs = jnp.concatenate([x1.T, jnp.ones((1, nb), _BF)], axis=0)

    @pl.when(i == 0)
    def _():
        u_scr[...] = jnp.zeros(u_scr.shape, u_scr.dtype)

    u_scr[...] += jnp.dot(lhs, hb, preferred_element_type=jnp.float32)

    @pl.when(i == nsteps - 1)
    def _():
        hid = u_scr.shape[0] - 1
        de = u_scr[hid:, :]
        s = wrow_ref[...] / (de + _EPS)
        srow_ref[...] = s
        e1t = u_scr[:hid, :] * s
        e1_ref[...] = e1t.T.astype(_BF)


def _pass_b(hb_ref, e1_ref, isdv_ref, w2_ref, b2_ref, srow_ref,
            e2_ref, u_scr):
    i = pl.program_id(0)
    nsteps = pl.num_programs(0)
    hb = hb_ref[...]
    isdv = isdv_ref[...]
    out1 = jnp.dot(hb, e1_ref[...],
                   preferred_element_type=jnp.float32) * isdv
    h1 = jnp.maximum(out1, 0.0)
    xw2 = jnp.dot(h1, w2_ref[...],
                  preferred_element_type=jnp.float32) + b2_ref[...]
    x12 = (xw2 * isdv).astype(_BF)

    @pl.when(i == 0)
    def _():
        u_scr[...] = jnp.zeros(u_scr.shape, u_scr.dtype)

    u_scr[...] += jnp.dot(x12.T, hb, preferred_element_type=jnp.float32)

    @pl.when(i == nsteps - 1)
    def _():
        e2t = u_scr[...] * srow_ref[...]
        e2_ref[...] = e2t.T.astype(_BF)


def _pass_c(hb_ref, e2_ref, isdv_ref, wh_ref, bh_ref, y_ref):
    out2 = jnp.dot(hb_ref[...], e2_ref[...],
                   preferred_element_type=jnp.float32) * isdv_ref[...]
    h2 = jnp.maximum(out2, 0.0)
    y_ref[...] = jnp.dot(h2, wh_ref[...],
                         preferred_element_type=jnp.float32) + bh_ref[...]


def kernel(x, H, w, W1, b1, W2, b2, Wh, bh):
    n, d_in = x.shape
    m = H.shape[1]
    hid = W1.shape[1]
    d_out = Wh.shape[1]
    nb = 400 if n % 400 == 0 else n
    grid = (n // nb,)

    hbm = H.astype(_BF)
    wcol = w.astype(_BF).reshape(m, 1)
    wrow = w.reshape(1, m)
    b1r = b1.reshape(1, hid)
    b2r = b2.reshape(1, hid)
    bhr = bh.reshape(1, d_out)

    isdv, srow, e1 = pl.pallas_call(
        _pass_a,
        grid=grid,
        in_specs=[
            pl.BlockSpec((nb, d_in), lambda i: (i, 0)),
            pl.BlockSpec((nb, m), lambda i: (i, 0)),
            pl.BlockSpec((m, 1), lambda i: (0, 0)),
            pl.BlockSpec((1, m), lambda i: (0, 0)),
            pl.BlockSpec((d_in, hid), lambda i: (0, 0)),
            pl.BlockSpec((1, hid), lambda i: (0, 0)),
        ],
        out_specs=[
            pl.BlockSpec((nb, 1), lambda i: (i, 0)),
            pl.BlockSpec((1, m), lambda i: (0, 0)),
            pl.BlockSpec((m, hid), lambda i: (0, 0)),
        ],
        out_shape=[
            jax.ShapeDtypeStruct((n, 1), jnp.float32),
            jax.ShapeDtypeStruct((1, m), jnp.float32),
            jax.ShapeDtypeStruct((m, hid), _BF),
        ],
        scratch_shapes=[pltpu.VMEM((hid + 1, m), jnp.float32)],
        compiler_params=_CP,
    )(x, hbm, wcol, wrow, W1, b1r)

    e2 = pl.pallas_call(
        _pass_b,
        grid=grid,
        in_specs=[
            pl.BlockSpec((nb, m), lambda i: (i, 0)),
            pl.BlockSpec((m, hid), lambda i: (0, 0)),
            pl.BlockSpec((nb, 1), lambda i: (i, 0)),
            pl.BlockSpec((hid, hid), lambda i: (0, 0)),
            pl.BlockSpec((1, hid), lambda i: (0, 0)),
            pl.BlockSpec((1, m), lambda i: (0, 0)),
        ],
        out_specs=pl.BlockSpec((m, hid), lambda i: (0, 0)),
        out_shape=jax.ShapeDtypeStruct((m, hid), _BF),
        scratch_shapes=[pltpu.VMEM((hid, m), jnp.float32)],
        compiler_params=_CP,
    )(hbm, e1, isdv, W2, b2r, srow)

    y = pl.pallas_call(
        _pass_c,
        grid=grid,
        in_specs=[
            pl.BlockSpec((nb, m), lambda i: (i, 0)),
            pl.BlockSpec((m, hid), lambda i: (0, 0)),
            pl.BlockSpec((nb, 1), lambda i: (i, 0)),
            pl.BlockSpec((hid, d_out), lambda i: (0, 0)),
            pl.BlockSpec((1, d_out), lambda i: (0, 0)),
        ],
        out_specs=pl.BlockSpec((nb, d_out), lambda i: (i, 0)),
        out_shape=jax.ShapeDtypeStruct((n, d_out), jnp.float32),
        compiler_params=_CP,
    )(hbm, e2, isdv, Wh, bhr)

    return y


# speedup vs baseline: 1.0432x; 1.0432x over previous
import jax
import jax.numpy as jnp
from jax import lax
from jax.experimental import pallas as pl
from jax.experimental.pallas import tpu as pltpu

_EPS = 1e-12
_BF = jnp.bfloat16
_CP = pltpu.CompilerParams(dimension_semantics=("arbitrary",),
                           vmem_limit_bytes=60 * 1024 * 1024)


def _pass_a(x_ref, h_ref, wcol_ref, wrow_ref, w1_ref, b1_ref,
            isdv_ref, srow_ref, e1_ref, u_scr):
    i = pl.program_id(0)
    nsteps = pl.num_programs(0)
    nb = h_ref.shape[0]
    hb = h_ref[...]
    dv = jnp.dot(hb, wcol_ref[...],
                 preferred_element_type=jnp.float32)
    isdv = lax.rsqrt(dv + _EPS)
    isdv_ref[...] = isdv
    xw = jnp.dot(x_ref[...], w1_ref[...],
                 preferred_element_type=jnp.float32) + b1_ref[...]
    x1 = (xw * isdv).astype(_BF)
    lhs = jnp.concatenate([x1.T, jnp.ones((1, nb), _BF)], axis=0)

    @pl.when(i == 0)
    def _():
        u_scr[...] = jnp.zeros(u_scr.shape, u_scr.dtype)

    u_scr[...] += jnp.dot(lhs, hb, preferred_element_type=jnp.float32)

    @pl.when(i == nsteps - 1)
    def _():
        hid = u_scr.shape[0] - 1
        de = u_scr[hid:, :]
        s = wrow_ref[...] / (de + _EPS)
        srow_ref[...] = s
        e1t = u_scr[:hid, :] * s
        e1_ref[...] = e1t.T.astype(_BF)


def _pass_b(hb_ref, e1_ref, isdv_ref, w2_ref, b2_ref, srow_ref,
            e2_ref, u_scr):
    i = pl.program_id(0)
    nsteps = pl.num_programs(0)
    hb = hb_ref[...]
    isdv = isdv_ref[...]
    out1 = jnp.dot(hb, e1_ref[...],
                   preferred_element_type=jnp.float32) * isdv
    h1 = jnp.maximum(out1, 0.0)
    xw2 = jnp.dot(h1, w2_ref[...],
                  preferred_element_type=jnp.float32) + b2_ref[...]
    x12 = (xw2 * isdv).astype(_BF)

    @pl.when(i == 0)
    def _():
        u_scr[...] = jnp.zeros(u_scr.shape, u_scr.dtype)

    u_scr[...] += jnp.dot(x12.T, hb, preferred_element_type=jnp.float32)

    @pl.when(i == nsteps - 1)
    def _():
        e2t = u_scr[...] * srow_ref[...]
        e2_ref[...] = e2t.T.astype(_BF)


def _pass_c(hb_ref, e2_ref, isdv_ref, wh_ref, bh_ref, y_ref):
    out2 = jnp.dot(hb_ref[...], e2_ref[...],
                   preferred_element_type=jnp.float32) * isdv_ref[...]
    h2 = jnp.maximum(out2, 0.0)
    y_ref[...] = jnp.dot(h2, wh_ref[...],
                         preferred_element_type=jnp.float32) + bh_ref[...]


def kernel(x, H, w, W1, b1, W2, b2, Wh, bh):
    n, d_in = x.shape
    m = H.shape[1]
    hid = W1.shape[1]
    d_out = Wh.shape[1]
    nb = 1000 if n % 1000 == 0 else n
    grid = (n // nb,)

    hbm = H.astype(_BF)
    wcol = w.astype(_BF).reshape(m, 1)
    wrow = w.reshape(1, m)
    b1r = b1.reshape(1, hid)
    b2r = b2.reshape(1, hid)
    bhr = bh.reshape(1, d_out)

    isdv, srow, e1 = pl.pallas_call(
        _pass_a,
        grid=grid,
        in_specs=[
            pl.BlockSpec((nb, d_in), lambda i: (i, 0)),
            pl.BlockSpec((nb, m), lambda i: (i, 0)),
            pl.BlockSpec((m, 1), lambda i: (0, 0)),
            pl.BlockSpec((1, m), lambda i: (0, 0)),
            pl.BlockSpec((d_in, hid), lambda i: (0, 0)),
            pl.BlockSpec((1, hid), lambda i: (0, 0)),
        ],
        out_specs=[
            pl.BlockSpec((nb, 1), lambda i: (i, 0)),
            pl.BlockSpec((1, m), lambda i: (0, 0)),
            pl.BlockSpec((m, hid), lambda i: (0, 0)),
        ],
        out_shape=[
            jax.ShapeDtypeStruct((n, 1), jnp.float32),
            jax.ShapeDtypeStruct((1, m), jnp.float32),
            jax.ShapeDtypeStruct((m, hid), _BF),
        ],
        scratch_shapes=[pltpu.VMEM((hid + 1, m), jnp.float32)],
        compiler_params=_CP,
    )(x, hbm, wcol, wrow, W1, b1r)

    e2 = pl.pallas_call(
        _pass_b,
        grid=grid,
        in_specs=[
            pl.BlockSpec((nb, m), lambda i: (i, 0)),
            pl.BlockSpec((m, hid), lambda i: (0, 0)),
            pl.BlockSpec((nb, 1), lambda i: (i, 0)),
            pl.BlockSpec((hid, hid), lambda i: (0, 0)),
            pl.BlockSpec((1, hid), lambda i: (0, 0)),
            pl.BlockSpec((1, m), lambda i: (0, 0)),
        ],
        out_specs=pl.BlockSpec((m, hid), lambda i: (0, 0)),
        out_shape=jax.ShapeDtypeStruct((m, hid), _BF),
        scratch_shapes=[pltpu.VMEM((hid, m), jnp.float32)],
        compiler_params=_CP,
    )(hbm, e1, isdv, W2, b2r, srow)

    y = pl.pallas_call(
        _pass_c,
        grid=grid,
        in_specs=[
            pl.BlockSpec((nb, m), lambda i: (i, 0)),
            pl.BlockSpec((m, hid), lambda i: (0, 0)),
            pl.BlockSpec((nb, 1), lambda i: (i, 0)),
            pl.BlockSpec((hid, d_out), lambda i: (0, 0)),
            pl.BlockSpec((1, d_out), lambda i: (0, 0)),
        ],
        out_specs=pl.BlockSpec((nb, d_out), lambda i: (i, 0)),
        out_shape=jax.ShapeDtypeStruct((n, d_out), jnp.float32),
        compiler_params=_CP,
    )(hbm, e2, isdv, Wh, bhr)

    return y
